# Initial kernel scaffold; baseline (speedup 1.0000x reference)
import jax
import jax.numpy as jnp
from jax.experimental import pallas as pl


def kernel(x, queries, ln_q_g, ln_q_b, ln_kv_g, ln_kv_b, w_in, b_in, w_out, b_out, ln_o_g, ln_o_b, w1, b1, w2, b2, codebook):
    raise NotImplementedError("write your pallas kernel here")


# speedup vs baseline: 1.0703x; 1.0703x over previous
import jax
import jax.numpy as jnp
import numpy as np
from jax import lax
from jax.experimental import pallas as pl
from jax.experimental.pallas import tpu as pltpu
from jax.experimental.pallas import tpu_sc as plsc

B, L, D, H, NT, K = 8, 2048, 1024, 16, 32, 1024
DH = D // H
R = B * NT

_SC_CORES = 2
_SC_SUBCORES = 16
_NW = _SC_CORES * _SC_SUBCORES
_ROWS_PER_W = R // _NW


def _lnorm(x, g, b):
    m = jnp.mean(x, axis=1, keepdims=True)
    v = jnp.mean((x - m) ** 2, axis=1, keepdims=True)
    return (x - m) / jnp.sqrt(v + 1e-5) * g + b


def _split(a):
    hi = a.astype(jnp.bfloat16)
    lo = (a - hi.astype(jnp.float32)).astype(jnp.bfloat16)
    return hi, lo


def _dot3(a_parts, b_parts, dims):
    (ah, al), (bh, bl) = a_parts, b_parts
    d = lambda x, y: lax.dot_general(x, y, (dims, ((), ())),
                                     preferred_element_type=jnp.float32)
    return d(ah, bh) + (d(ah, bl) + d(al, bh))


_NT_DIMS = ((1,), (1,))
_NN_DIMS = ((1,), (0,))


def _mm_t(a, b_mat):
    return _dot3(_split(a), _split(b_mat), _NT_DIMS)


def _mm(a, b_mat):
    return _dot3(_split(a), _split(b_mat), _NN_DIMS)


def _attn_body(x_ref, queries_ref, lnqg_ref, lnqb_ref, lnkvg_ref, lnkvb_ref,
               wq_ref, wk_ref, wv_ref, bq_ref, bv_ref, wo_ref, bo_ref,
               q1_ref, kvnh_s, kvnl_s, qth_s, qtl_s):
    b = pl.program_id(0)

    @pl.when(b == 0)
    def _():
        qn = _lnorm(queries_ref[...], lnqg_ref[...], lnqb_ref[...])
        qf = _mm_t(qn, wq_ref[...]) + bq_ref[...]
        for h in range(H):
            qt_h = _mm(qf[:, h * DH:(h + 1) * DH],
                       wk_ref[h * DH:(h + 1) * DH, :])
            hi, lo = _split(qt_h)
            qth_s[h * NT:(h + 1) * NT, :] = hi
            qtl_s[h * NT:(h + 1) * NT, :] = lo

    kvn_hi, kvn_lo = _split(_lnorm(x_ref[0], lnkvg_ref[...], lnkvb_ref[...]))
    kvnh_s[...] = kvn_hi
    kvnl_s[...] = kvn_lo
    kvn_parts = (kvnh_s[...], kvnl_s[...])
    s = _dot3((qth_s[...], qtl_s[...]), kvn_parts, _NT_DIMS) * (1.0 / np.sqrt(DH))
    s = s - jnp.max(s, axis=1, keepdims=True)
    e = jnp.exp(s)
    att = e / jnp.sum(e, axis=1, keepdims=True)
    ctx = _dot3(_split(att), kvn_parts, _NN_DIMS)
    ctx_parts = _split(ctx)
    wv_parts = _split(wv_ref[...])
    o = jnp.concatenate(
        [_dot3(tuple(p[h * NT:(h + 1) * NT, :] for p in ctx_parts),
               tuple(p[h * DH:(h + 1) * DH, :] for p in wv_parts), _NT_DIMS)
         for h in range(H)], axis=1) + bv_ref[...]
    q1_ref[0] = queries_ref[...] + _mm_t(o, wo_ref[...]) + bo_ref[...]


def _mlp_vq_body(q1_ref, lnog_ref, lnob_ref, w1_ref, b1_ref, w2_ref, b2_ref,
                 cb_ref, idx_ref, commit_ref):
    q1 = q1_ref[...]
    hn = _lnorm(q1, lnog_ref[...], lnob_ref[...])
    h1 = _mm_t(hn, w1_ref[...]) + b1_ref[...]
    h1 = 0.5 * h1 * (1.0 + lax.erf(h1 * (1.0 / np.sqrt(2.0))))
    z = q1 + _mm_t(h1, w2_ref[...]) + b2_ref[...]
    cb = cb_ref[...]
    cb_parts = _split(cb)
    cb2 = _mm_t(jnp.ones((1, D), jnp.float32), cb * cb)
    zsq = jnp.sum(z * z, axis=1, keepdims=True)
    d2 = (zsq - 2.0 * _dot3(_split(z), cb_parts, _NT_DIMS)) + cb2
    m = jnp.min(d2, axis=1, keepdims=True)
    ii = lax.broadcasted_iota(jnp.int32, (R, K), 1)
    idx_ref[...] = jnp.min(jnp.where(d2 == m, ii, jnp.int32(2 ** 30)),
                           axis=1, keepdims=True)
    commit_ref[0, 0] = (0.25 / (R * D)) * jnp.sum(m)


def _gather_body(idx_hbm, cb_hbm, zq_hbm, idx_v, rows_v, sem):
    wid = lax.axis_index("s") * _SC_CORES + lax.axis_index("c")
    base = wid * _ROWS_PER_W
    pltpu.sync_copy(idx_hbm.at[pl.ds(base, _ROWS_PER_W)], idx_v)
    pltpu.async_copy(cb_hbm.at[idx_v], rows_v, sem).wait()
    pltpu.sync_copy(rows_v, zq_hbm.at[pl.ds(base, _ROWS_PER_W)])


def kernel(x, queries, ln_q_g, ln_q_b, ln_kv_g, ln_kv_b, w_in, b_in, w_out,
           b_out, ln_o_g, ln_o_b, w1, b1, w2, b2, codebook):
    wq, wk, wv = w_in[:D], w_in[D:2 * D], w_in[2 * D:]
    bq, bv = b_in[None, :D], b_in[None, 2 * D:]
    row = lambda v: v[None, :]

    vspec = pl.BlockSpec((1, D), lambda b: (0, 0))
    mspec = pl.BlockSpec((D, D), lambda b: (0, 0))

    q1 = pl.pallas_call(
        _attn_body,
        grid=(B,),
        in_specs=[
            pl.BlockSpec((1, L, D), lambda b: (b, 0, 0)),
            pl.BlockSpec((NT, D), lambda b: (0, 0)),
            vspec, vspec, vspec, vspec,
            mspec, mspec, mspec,
            vspec, vspec,
            mspec,
            vspec,
        ],
        out_specs=pl.BlockSpec((1, NT, D), lambda b: (b, 0, 0)),
        out_shape=jax.ShapeDtypeStruct((B, NT, D), jnp.float32),
        scratch_shapes=[
            pltpu.VMEM((L, D), jnp.bfloat16),
            pltpu.VMEM((L, D), jnp.bfloat16),
            pltpu.VMEM((H * NT, D), jnp.bfloat16),
            pltpu.VMEM((H * NT, D), jnp.bfloat16),
        ],
        compiler_params=pltpu.CompilerParams(vmem_limit_bytes=100 * 2**20),
    )(x, queries, row(ln_q_g), row(ln_q_b), row(ln_kv_g), row(ln_kv_b),
      wq, wk, wv, bq, bv, w_out, row(b_out))

    idx, commit = pl.pallas_call(
        _mlp_vq_body,
        in_specs=[
            pl.BlockSpec((R, D), lambda: (0, 0)),
            pl.BlockSpec((1, D), lambda: (0, 0)),
            pl.BlockSpec((1, D), lambda: (0, 0)),
            pl.BlockSpec((4 * D, D), lambda: (0, 0)),
            pl.BlockSpec((1, 4 * D), lambda: (0, 0)),
            pl.BlockSpec((D, 4 * D), lambda: (0, 0)),
            pl.BlockSpec((1, D), lambda: (0, 0)),
            pl.BlockSpec((K, D), lambda: (0, 0)),
        ],
        out_specs=[
            pl.BlockSpec((R, 1), lambda: (0, 0)),
            pl.BlockSpec(memory_space=pltpu.SMEM),
        ],
        out_shape=[
            jax.ShapeDtypeStruct((R, 1), jnp.int32),
            jax.ShapeDtypeStruct((1, 1), jnp.float32),
        ],
        compiler_params=pltpu.CompilerParams(vmem_limit_bytes=100 * 2**20),
    )(q1.reshape(R, D), row(ln_o_g), row(ln_o_b), w1, row(b1), w2, row(b2),
      codebook)

    idx = idx.reshape(R)
    zq = pl.kernel(
        _gather_body,
        out_type=jax.ShapeDtypeStruct((R, D), jnp.float32),
        mesh=plsc.VectorSubcoreMesh(core_axis_name="c", subcore_axis_name="s"),
        scratch_types=[
            pltpu.VMEM((_ROWS_PER_W,), jnp.int32),
            pltpu.VMEM((_ROWS_PER_W, D), jnp.float32),
            pltpu.SemaphoreType.DMA,
        ],
    )(idx, codebook)

    return zq.reshape(B, NT, D), idx.reshape(B, NT), commit.reshape(())
